# Initial kernel scaffold; baseline (speedup 1.0000x reference)
import jax
import jax.numpy as jnp
from jax.experimental import pallas as pl


def kernel(inputs, params, originalInput):
    raise NotImplementedError("write your pallas kernel here")


# speedup vs baseline: 5.9331x; 5.9331x over previous
import functools

import jax
import jax.numpy as jnp
import numpy as np
from jax.experimental import pallas as pl
from jax.experimental.pallas import tpu as pltpu

B = 16
H = 224
W = 224
R = 14
NSEG = R * R
NP = 208
CH = 192


def _stem_kernel(x_ref, slic_ref, feat_ref):
    x = x_ref[0]

    r224 = jax.lax.broadcasted_iota(jnp.int32, (H, R), 0)
    c14 = jax.lax.broadcasted_iota(jnp.int32, (H, R), 1)
    P = (r224 // 16 == c14).astype(jnp.float32)
    C = (r224 == 16 * c14 + 7).astype(jnp.float32)
    r14 = jax.lax.broadcasted_iota(jnp.int32, (R, H), 0)
    c224 = jax.lax.broadcasted_iota(jnp.int32, (R, H), 1)
    PT = (c224 // 16 == r14).astype(jnp.float32)
    CT = (c224 == 16 * r14 + 7).astype(jnp.float32)

    dot = functools.partial(jnp.dot, preferred_element_type=jnp.float32)

    gy = jax.lax.broadcasted_iota(jnp.int32, (R, R), 0).astype(jnp.float32)
    gx = jax.lax.broadcasted_iota(jnp.int32, (R, R), 1).astype(jnp.float32)
    feat_ref[0, 0] = 16.0 * gy + 7.5
    feat_ref[0, 1] = 16.0 * gx + 7.5

    for c in range(3):
        X = x[c]
        S1 = dot(dot(PT, X), P)
        S2 = dot(dot(PT, X * X), P)
        CTR = dot(dot(CT, X), C)
        mean = S1 * (1.0 / 256.0)
        var = (S2 - 256.0 * mean * mean) * (1.0 / 255.0)
        std = jnp.sqrt(jnp.maximum(var, 0.0))
        feat_ref[0, 2 + c] = mean
        feat_ref[0, 5 + c] = std
        feat_ref[0, 8 + c] = CTR

    for c in range(11, 16):
        feat_ref[0, c] = jnp.zeros((R, R), jnp.float32)

    ry = jax.lax.broadcasted_iota(jnp.int32, (H, W), 0)
    rx = jax.lax.broadcasted_iota(jnp.int32, (H, W), 1)
    slic_ref[0] = (ry // 16) * R + rx // 16


def _stem_pallas(inputs):
    slic, feat = pl.pallas_call(
        _stem_kernel,
        grid=(B,),
        in_specs=[pl.BlockSpec((1, 3, H, W), lambda i: (i, 0, 0, 0))],
        out_specs=[
            pl.BlockSpec((1, H, W), lambda i: (i, 0, 0)),
            pl.BlockSpec((1, 16, R, R), lambda i: (i, 0, 0, 0)),
        ],
        out_shape=[
            jax.ShapeDtypeStruct((B, H, W), jnp.int32),
            jax.ShapeDtypeStruct((B, 16, R, R), jnp.float32),
        ],
    )(inputs)
    return slic, feat


def _fold(p):
    return p['W'] * p['g'][:, None], p['b'] * p['g'] + p['be']


def _conv_bn_jnp(x, p, act=False):
    Wf, bf = _fold(p)
    y = jnp.einsum('bchw,oc->bohw', x, Wf) + bf[None, :, None, None]
    if act:
        y = jax.nn.relu(y)
    return y


def _mr_graph_conv_jnp(x, k, p):
    Bc, C, Hh, Ww = x.shape
    N = Hh * Ww
    xf = x.reshape(Bc, C, N).transpose(0, 2, 1)
    sq = jnp.sum(xf * xf, axis=-1)
    dist = sq[:, :, None] + sq[:, None, :] - 2.0 * jnp.einsum('bnc,bmc->bnm', xf, xf)
    _, idx = jax.lax.top_k(-dist, k)
    xj = jax.vmap(lambda f, i: f[i])(xf, idx)
    rel = jnp.max(xj - xf[:, :, None, :], axis=2)
    cat = jnp.concatenate([xf, rel], axis=-1).transpose(0, 2, 1).reshape(Bc, 2 * C, Hh, Ww)
    return _conv_bn_jnp(cat, p, act=True)


def kernel(inputs, params, originalInput):
    x_slic, feat = _stem_pallas(inputs)

    x = feat[:, :11]
    for i, p in enumerate(params['stem']):
        x = _conv_bn_jnp(x, p, act=(i < 4))

    num_knn = [9, 18]
    featMaps = []
    for i, blk in enumerate(params['blocks']):
        shortcut = x
        y = _conv_bn_jnp(x, blk['fc1'], act=False)
        y = _mr_graph_conv_jnp(y, num_knn[i], blk['mr'])
        y = _conv_bn_jnp(y, blk['fc2'], act=False)
        x = y + shortcut
        shortcut = x
        y = _conv_bn_jnp(x, blk['ffn1'], act=True)
        y = _conv_bn_jnp(y, blk['ffn2'], act=False)
        x = y + shortcut
        featMaps.append(x)

    x = jnp.mean(x, axis=(2, 3), keepdims=True)
    x = _conv_bn_jnp(x, params['pred1'], act=True)
    pred = jnp.einsum('bchw,oc->bohw', x, params['pred2']['W']) + params['pred2']['b'][None, :, None, None]
    pred = pred[:, :, 0, 0]
    return pred, x_slic, tuple(featMaps)


# speedup vs baseline: 42.3121x; 7.1316x over previous
import functools

import jax
import jax.numpy as jnp
from jax.experimental import pallas as pl
from jax.experimental.pallas import tpu as pltpu

B = 16
H = 224
W = 224
R = 14
NSEG = R * R
NP = 208
NT = B * NP
CH = 192
INF = 3.0e38

_dot = functools.partial(jnp.dot, preferred_element_type=jnp.float32)
_dot_hi = functools.partial(jnp.dot, preferred_element_type=jnp.float32,
                            precision=jax.lax.Precision.HIGHEST)
_dot_sel = _dot_hi


def _stem_kernel(x_ref, slic_ref, feat_ref):
    x = x_ref[0]

    r224 = jax.lax.broadcasted_iota(jnp.int32, (H, R), 0)
    c14 = jax.lax.broadcasted_iota(jnp.int32, (H, R), 1)
    P = (r224 // 16 == c14).astype(jnp.float32)
    C = (r224 == 16 * c14 + 7).astype(jnp.float32)
    r14 = jax.lax.broadcasted_iota(jnp.int32, (R, H), 0)
    c224 = jax.lax.broadcasted_iota(jnp.int32, (R, H), 1)
    PT = (c224 // 16 == r14).astype(jnp.float32)
    CT = (c224 == 16 * r14 + 7).astype(jnp.float32)

    gy = jax.lax.broadcasted_iota(jnp.int32, (R, R), 0).astype(jnp.float32)
    gx = jax.lax.broadcasted_iota(jnp.int32, (R, R), 1).astype(jnp.float32)
    feat_ref[0, 0] = 16.0 * gy + 7.5
    feat_ref[0, 1] = 16.0 * gx + 7.5

    for c in range(3):
        X = x[c]
        S1 = _dot_hi(_dot_hi(PT, X), P)
        S2 = _dot_hi(_dot_hi(PT, X * X), P)
        CTR = _dot_hi(_dot_hi(CT, X), C)
        mean = S1 * (1.0 / 256.0)
        var = (S2 - 256.0 * mean * mean) * (1.0 / 255.0)
        std = jnp.sqrt(jnp.maximum(var, 0.0))
        feat_ref[0, 2 + c] = mean
        feat_ref[0, 5 + c] = std
        feat_ref[0, 8 + c] = CTR

    for c in range(11, 16):
        feat_ref[0, c] = jnp.zeros((R, R), jnp.float32)

    ry = jax.lax.broadcasted_iota(jnp.int32, (H, W), 0)
    rx = jax.lax.broadcasted_iota(jnp.int32, (H, W), 1)
    slic_ref[0] = (ry // 16) * R + rx // 16


def _stem_pallas(inputs):
    return pl.pallas_call(
        _stem_kernel,
        grid=(B,),
        in_specs=[pl.BlockSpec((1, 3, H, W), lambda i: (i, 0, 0, 0))],
        out_specs=[
            pl.BlockSpec((1, H, W), lambda i: (i, 0, 0)),
            pl.BlockSpec((1, 16, R, R), lambda i: (i, 0, 0, 0)),
        ],
        out_shape=[
            jax.ShapeDtypeStruct((B, H, W), jnp.int32),
            jax.ShapeDtypeStruct((B, 16, R, R), jnp.float32),
        ],
    )(inputs)


def _grapher_block(x, k, dist_ref, maxf_ref,
                   wfc1, bfc1, wmra, wmrb, bmr, wfc2, bfc2, w1, b1, w2, b2):
    y = _dot(x, wfc1) + bfc1

    row_np = jax.lax.broadcasted_iota(jnp.int32, (NP, NP), 0)
    lane_np = jax.lax.broadcasted_iota(jnp.int32, (NP, NP), 1)
    eye = (row_np == lane_np).astype(jnp.float32)
    for i in range(B):
        yi = y[i * NP:(i + 1) * NP, :]
        G = jax.lax.dot_general(yi, yi, (((1,), (1,)), ((), ())),
                                preferred_element_type=jnp.float32)
        sq_col = jnp.sum(yi * yi, axis=1, keepdims=True)
        sq_row = jnp.sum(eye * sq_col, axis=0, keepdims=True)
        d = (sq_col + sq_row) - 2.0 * G
        d = jnp.where(lane_np[0:1] >= NSEG, INF, d)
        dist_ref[i * NP:(i + 1) * NP, :] = d

    maxf_ref[...] = jnp.full((NT, CH), -INF, jnp.float32)
    lane_all = jax.lax.broadcasted_iota(jnp.int32, (NT, NP), 1)

    def body(_, carry):
        d = dist_ref[...]
        minv = jnp.min(d, axis=1, keepdims=True)
        cand = jnp.where(d <= minv, lane_all, 1000)
        jsel = jnp.min(cand, axis=1, keepdims=True)
        oh = lane_all == jsel
        dist_ref[...] = jnp.where(oh, INF, d)
        ohf = oh.astype(jnp.float32)
        for i in range(B):
            sl = pl.ds(i * NP, NP)
            sel = _dot_sel(ohf[i * NP:(i + 1) * NP, :], y[i * NP:(i + 1) * NP, :])
            maxf_ref[sl, :] = jnp.maximum(maxf_ref[sl, :], sel)
        return carry

    jax.lax.fori_loop(0, k, body, 0)

    rel = maxf_ref[...] - y
    y = jax.nn.relu(_dot(y, wmra) + _dot(rel, wmrb) + bmr)
    x = _dot(y, wfc2) + bfc2 + x
    h = jax.nn.relu(_dot(x, w1) + b1)
    return _dot(h, w2) + b2 + x


def _pc2_kernel(feats_ref,
                s0w, s0b, s1w, s1b, s2w, s2b, s3w, s3b, s4w, s4b,
                wfc1, bfc1, wmra, wmrb, bmr, wfc2, bfc2, wf1, bf1, wf2, bf2,
                fm1_ref, dist_ref, maxf_ref):
    x = feats_ref[...]
    for wref, bref, act in ((s0w, s0b, True), (s1w, s1b, True),
                            (s2w, s2b, True), (s3w, s3b, True),
                            (s4w, s4b, False)):
        x = _dot(x, wref[...]) + bref[...]
        if act:
            x = jax.nn.relu(x)
    fm1_ref[...] = _grapher_block(
        x, 9, dist_ref, maxf_ref,
        wfc1[...], bfc1[...], wmra[...], wmrb[...], bmr[...],
        wfc2[...], bfc2[...], wf1[...], bf1[...], wf2[...], bf2[...])


def _pc3_kernel(fm1_ref,
                wfc1, bfc1, wmra, wmrb, bmr, wfc2, bfc2, wf1, bf1, wf2, bf2,
                wp1, bp1, wp2, bp2,
                fm2_ref, pred_ref, dist_ref, maxf_ref):
    x = _grapher_block(
        fm1_ref[...], 18, dist_ref, maxf_ref,
        wfc1[...], bfc1[...], wmra[...], wmrb[...], bmr[...],
        wfc2[...], bfc2[...], wf1[...], bf1[...], wf2[...], bf2[...])
    fm2_ref[...] = x

    srow = jax.lax.broadcasted_iota(jnp.int32, (B, NT), 0)
    scol = jax.lax.broadcasted_iota(jnp.int32, (B, NT), 1)
    S = ((scol // NP == srow) & (scol - (scol // NP) * NP < NSEG)).astype(
        jnp.float32) * (1.0 / NSEG)
    pooled = _dot(S, x)
    h = jax.nn.relu(_dot(pooled, wp1[...]) + bp1[...])
    pred_ref[...] = _dot(h, wp2[...]) + bp2[...]


def _fold_t(p, pad_in=None):
    wt = (p['W'] * p['g'][:, None]).T
    b = (p['b'] * p['g'] + p['be'])[None, :]
    if pad_in is not None and pad_in != wt.shape[0]:
        wt = jnp.pad(wt, ((0, pad_in - wt.shape[0]), (0, 0)))
    return wt, b


def _to_map(fm):
    return fm.reshape(B, NP, CH)[:, :NSEG].transpose(0, 2, 1).reshape(B, CH, R, R)


def kernel(inputs, params, originalInput):
    x_slic, feat = _stem_pallas(inputs)
    f = feat.reshape(B, 16, NSEG).transpose(0, 2, 1)
    f = jnp.pad(f, ((0, 0), (0, NP - NSEG), (0, 0))).reshape(NT, 16)

    stem_w = [_fold_t(p, pad_in=16 if i == 0 else None)
              for i, p in enumerate(params['stem'])]

    def block_w(blk):
        wfc1, bfc1 = _fold_t(blk['fc1'])
        wmr, bmr = _fold_t(blk['mr'])
        wfc2, bfc2 = _fold_t(blk['fc2'])
        wf1, bf1 = _fold_t(blk['ffn1'])
        wf2, bf2 = _fold_t(blk['ffn2'])
        return (wfc1, bfc1, wmr[:CH], wmr[CH:], bmr, wfc2, bfc2,
                wf1, bf1, wf2, bf2)

    bw1 = block_w(params['blocks'][0])
    bw2 = block_w(params['blocks'][1])
    wp1, bp1 = _fold_t(params['pred1'])
    wp2 = params['pred2']['W'].T
    bp2 = params['pred2']['b'][None, :]

    fm1 = pl.pallas_call(
        _pc2_kernel,
        out_shape=jax.ShapeDtypeStruct((NT, CH), jnp.float32),
        scratch_shapes=[pltpu.VMEM((NT, NP), jnp.float32),
                        pltpu.VMEM((NT, CH), jnp.float32)],
    )(f, *[a for wb in stem_w for a in wb], *bw1)

    fm2, pred = pl.pallas_call(
        _pc3_kernel,
        out_shape=[jax.ShapeDtypeStruct((NT, CH), jnp.float32),
                   jax.ShapeDtypeStruct((B, 1000), jnp.float32)],
        scratch_shapes=[pltpu.VMEM((NT, NP), jnp.float32),
                        pltpu.VMEM((NT, CH), jnp.float32)],
    )(fm1, *bw2, wp1, bp1, wp2, bp2)

    return pred, x_slic, (_to_map(fm1), _to_map(fm2))


# speedup vs baseline: 63.1140x; 1.4916x over previous
import functools

import jax
import jax.numpy as jnp
from jax.experimental import pallas as pl
from jax.experimental.pallas import tpu as pltpu

B = 16
H = 224
W = 224
R = 14
NSEG = R * R
NP = 208
NT = B * NP
CH = 192
INF = 3.0e38

_DNT = (((1,), (1,)), ((), ()))
_DN0 = (((0,), (0,)), ((), ()))

_dot = functools.partial(jnp.dot, preferred_element_type=jnp.float32)
_dot_hi = functools.partial(jnp.dot, preferred_element_type=jnp.float32,
                            precision=jax.lax.Precision.HIGHEST)


def _dg(a, b, dn):
    return jax.lax.dot_general(a, b, dn, preferred_element_type=jnp.float32)


def _split_hi_lo(v):
    hi = v.astype(jnp.bfloat16).astype(jnp.float32)
    return hi, v - hi


def _dotT_exact(a, b):
    b_hi, b_lo = _split_hi_lo(b)
    return _dg(a, b_hi, _DN0) + _dg(a, b_lo, _DN0)


def _conv(x, p, act):
    wref, gref, bref, beref = p
    g = gref[...]
    y = _dg(x, wref[...], _DNT) * g + (bref[...] * g + beref[...])
    return jax.nn.relu(y) if act else y


def _stem_kernel(x_ref, slic_ref, feat_ref):
    x = x_ref[0]

    r224 = jax.lax.broadcasted_iota(jnp.int32, (H, R), 0)
    c14 = jax.lax.broadcasted_iota(jnp.int32, (H, R), 1)
    P = (r224 // 16 == c14).astype(jnp.float32)
    C = (r224 == 16 * c14 + 7).astype(jnp.float32)

    gy = jax.lax.broadcasted_iota(jnp.int32, (R, R), 0).astype(jnp.float32)
    gx = jax.lax.broadcasted_iota(jnp.int32, (R, R), 1).astype(jnp.float32)
    feat_ref[0, 0] = 16.0 * gy + 7.5
    feat_ref[0, 1] = 16.0 * gx + 7.5

    for c in range(3):
        X = x[c]
        Xr = X.reshape(R, 16, W)
        S1r = jnp.sum(Xr, axis=1)
        S2r = jnp.sum(Xr * Xr, axis=1)
        ctr_rows = jax.lax.slice_in_dim(Xr, 7, 8, axis=1).reshape(R, W)
        S1 = _dot_hi(S1r, P)
        S2 = _dot_hi(S2r, P)
        CTR = _dot_hi(ctr_rows, C)
        mean = S1 * (1.0 / 256.0)
        var = (S2 - 256.0 * mean * mean) * (1.0 / 255.0)
        std = jnp.sqrt(jnp.maximum(var, 0.0))
        feat_ref[0, 2 + c] = mean
        feat_ref[0, 5 + c] = std
        feat_ref[0, 8 + c] = CTR

    for c in range(11, 16):
        feat_ref[0, c] = jnp.zeros((R, R), jnp.float32)

    ry = jax.lax.broadcasted_iota(jnp.int32, (H, W), 0)
    rx = jax.lax.broadcasted_iota(jnp.int32, (H, W), 1)
    slic_ref[0] = (ry // 16) * R + rx // 16


def _stem_pallas(inputs):
    return pl.pallas_call(
        _stem_kernel,
        grid=(B,),
        in_specs=[pl.BlockSpec((1, 3, H, W), lambda i: (i, 0, 0, 0))],
        out_specs=[
            pl.BlockSpec((1, H, W), lambda i: (i, 0, 0)),
            pl.BlockSpec((1, 16, R, R), lambda i: (i, 0, 0, 0)),
        ],
        out_shape=[
            jax.ShapeDtypeStruct((B, H, W), jnp.int32),
            jax.ShapeDtypeStruct((B, 16, R, R), jnp.float32),
        ],
    )(inputs)


def _grapher_block(x, k, dist_ref, maxf_ref, fc1, mr, fc2, ffn1, ffn2):
    y = _conv(x, fc1, act=False)

    row_np = jax.lax.broadcasted_iota(jnp.int32, (NP, NP), 0)
    lane_np = jax.lax.broadcasted_iota(jnp.int32, (NP, NP), 1)
    eye = (row_np == lane_np).astype(jnp.float32)
    for i in range(B):
        yi = y[i * NP:(i + 1) * NP, :]
        G = _dg(yi, yi, _DNT)
        sq_col = jnp.sum(yi * yi, axis=1, keepdims=True)
        sq_row = jnp.sum(eye * sq_col, axis=0, keepdims=True)
        d = (sq_col + sq_row) - 2.0 * G
        d = jnp.where(row_np[:, 0:1] >= NSEG, INF, d)
        dist_ref[:, i * NP:(i + 1) * NP] = d

    maxf_ref[...] = jnp.full((NT, CH), -INF, jnp.float32)
    sub_all = jax.lax.broadcasted_iota(jnp.int32, (NP, NT), 0)
    y_hi, y_lo = _split_hi_lo(y)

    def body(_, carry):
        d = dist_ref[...]
        minv = jnp.min(d, axis=0, keepdims=True)
        cand = jnp.where(d == minv, sub_all, 1000)
        jsel = jnp.min(cand, axis=0, keepdims=True)
        oh = sub_all == jsel
        dist_ref[...] = jnp.where(oh, INF, d)
        ohf = oh.astype(jnp.float32)
        for i in range(B):
            sl = slice(i * NP, (i + 1) * NP)
            ohi = ohf[:, sl]
            sel = _dg(ohi, y_hi[sl, :], _DN0) + _dg(ohi, y_lo[sl, :], _DN0)
            maxf_ref[sl, :] = jnp.maximum(maxf_ref[sl, :], sel)
        return carry

    jax.lax.fori_loop(0, k, body, 0)

    rel = maxf_ref[...] - y
    wmr, gmr, bmr, bemr = mr
    wmrv = wmr[...]
    gv = gmr[...]
    y = jax.nn.relu((_dg(y, wmrv[:, :CH], _DNT) + _dg(rel, wmrv[:, CH:], _DNT))
                    * gv + (bmr[...] * gv + bemr[...]))
    x = _conv(y, fc2, act=False) + x
    h = _conv(x, ffn1, act=True)
    return _conv(h, ffn2, act=False) + x


def _store_fmT(x, eye_np, fmt_ref):
    for i in range(B):
        xT = _dotT_exact(x[i * NP:(i + 1) * NP, :], eye_np)
        fmt_ref[i] = xT[:, :NSEG]


def _pc2_kernel(feats_ref, *refs):
    it = iter(refs)

    def take(n):
        return tuple(next(it) for _ in range(n))

    stem = [take(4) for _ in range(5)]
    blk1 = [take(4) for _ in range(5)]
    blk2 = [take(4) for _ in range(5)]
    pred1 = take(4)
    w2ref, b2ref = take(2)
    fm1t_ref, fm2t_ref, pred_ref, dist_ref, maxf_ref = take(5)

    x = feats_ref[...]
    for li, p in enumerate(stem):
        x = _conv(x, p, act=(li < 4))

    row_np = jax.lax.broadcasted_iota(jnp.int32, (NP, NP), 0)
    lane_np = jax.lax.broadcasted_iota(jnp.int32, (NP, NP), 1)
    eye_np = (row_np == lane_np).astype(jnp.float32)

    x = _grapher_block(x, 9, dist_ref, maxf_ref, *blk1)
    _store_fmT(x, eye_np, fm1t_ref)
    x = _grapher_block(x, 18, dist_ref, maxf_ref, *blk2)
    _store_fmT(x, eye_np, fm2t_ref)

    srow = jax.lax.broadcasted_iota(jnp.int32, (NT, B), 1)
    scol = jax.lax.broadcasted_iota(jnp.int32, (NT, B), 0)
    S = ((scol // NP == srow) & (scol - (scol // NP) * NP < NSEG)).astype(
        jnp.float32)
    pooled = _dotT_exact(S, x) * (1.0 / NSEG)
    h = _conv(pooled, pred1, act=True)
    pred_ref[...] = _dg(h, w2ref[...], _DNT) + b2ref[...]


def kernel(inputs, params, originalInput):
    x_slic, feat = _stem_pallas(inputs)
    f = feat.reshape(B, 16, NSEG).transpose(0, 2, 1)
    f = jnp.pad(f, ((0, 0), (0, NP - NSEG), (0, 0))).reshape(NT, 16)

    def conv_args(p, pad_in=None):
        w = p['W']
        if pad_in is not None and pad_in != w.shape[1]:
            w = jnp.pad(w, ((0, 0), (0, pad_in - w.shape[1])))
        return [w, p['g'][None, :], p['b'][None, :], p['be'][None, :]]

    args = []
    for i, p in enumerate(params['stem']):
        args += conv_args(p, pad_in=16 if i == 0 else None)
    for blk in params['blocks']:
        for name in ('fc1', 'mr', 'fc2', 'ffn1', 'ffn2'):
            args += conv_args(blk[name])
    args += conv_args(params['pred1'])
    args += [params['pred2']['W'], params['pred2']['b'][None, :]]

    fm1t, fm2t, pred = pl.pallas_call(
        _pc2_kernel,
        out_shape=[jax.ShapeDtypeStruct((B, CH, NSEG), jnp.float32),
                   jax.ShapeDtypeStruct((B, CH, NSEG), jnp.float32),
                   jax.ShapeDtypeStruct((B, 1000), jnp.float32)],
        scratch_shapes=[pltpu.VMEM((NP, NT), jnp.float32),
                        pltpu.VMEM((NT, CH), jnp.float32)],
    )(f, *args)

    return (pred, x_slic,
            (fm1t.reshape(B, CH, R, R), fm2t.reshape(B, CH, R, R)))
